# baseline (speedup 1.0000x reference)
import functools

import jax
import jax.numpy as jnp
from jax import lax
from jax.experimental import pallas as pl
from jax.experimental.pallas import tpu as pltpu
from jax.experimental.pallas import tpu_sc as plsc

N = 100000
D = 128
NUM_CORES = 2
NUM_SUBCORES = 16
NUM_WORKERS = NUM_CORES * NUM_SUBCORES
PER_W = 3136
LAST_BASE = N - PER_W
CHUNK = 224
NCHUNK = PER_W // CHUNK
NBUF = 3
V = 94
L = 16

_mesh = plsc.VectorSubcoreMesh(core_axis_name="c", subcore_axis_name="s")


@functools.partial(
    pl.kernel,
    mesh=_mesh,
    compiler_params=pltpu.CompilerParams(needs_layout_passes=False),
    out_type=jax.ShapeDtypeStruct((N, D), jnp.float32),
    scratch_types=[
        pltpu.VMEM((PER_W,), jnp.int32),
        pltpu.VMEM((NBUF, CHUNK, D), jnp.float32),
        pltpu.VMEM((V, D), jnp.float32),
        pltpu.VMEM_SHARED((V, D), jnp.float32),
        pltpu.SemaphoreType.DMA,
        pltpu.SemaphoreType.DMA,
        pltpu.SemaphoreType.DMA,
        pltpu.SemaphoreType.DMA,
        pltpu.SemaphoreType.DMA,
        pltpu.SemaphoreType.DMA,
        pltpu.SemaphoreType.DMA,
        pltpu.SemaphoreType.DMA,
    ],
)
def _emb_lookup(idx_hbm, table_hbm, out_hbm, idx_v, rows_v, table_l, table_sh,
                isem, tsem, gsem0, gsem1, gsem2, ssem0, ssem1, ssem2):
    gsems = (gsem0, gsem1, gsem2)
    ssems = (ssem0, ssem1, ssem2)
    sid = lax.axis_index("s")
    wid = sid * NUM_CORES + lax.axis_index("c")
    base = jnp.minimum(wid * PER_W, LAST_BASE)

    @pl.when(sid == 0)
    def _():
        pltpu.sync_copy(table_hbm, table_sh)

    tcp = pltpu.async_copy(table_hbm, table_l, tsem)
    pltpu.async_copy(idx_hbm.at[pl.ds(base, PER_W)], idx_v, isem).wait()
    tcp.wait()
    plsc.subcore_barrier()

    def dma_gather(j):
        return pltpu.async_copy(
            table_sh.at[idx_v.at[pl.ds(j * CHUNK, CHUNK)]],
            rows_v.at[j % NBUF],
            gsems[j % NBUF],
        )

    cols = [lax.iota(jnp.int32, 16) + 16 * c for c in range(D // L)]

    def compute_gather(j, b):
        def body(a, carry):
            ids = plsc.load_gather(
                idx_v, [jnp.full((L,), j * CHUNK + a, jnp.int32)])
            for c in range(D // L):
                val = plsc.load_gather(table_l, [ids, cols[c]])
                rows_v[b, a, pl.ds(L * c, L)] = val
            return carry
        lax.fori_loop(0, CHUNK, body, 0)

    waited = set()
    stores = {}

    def buf_free(j):
        k = j - NBUF
        if k >= 0 and k not in waited:
            stores[k].wait()
            waited.add(k)

    gathers = {0: dma_gather(0)}
    for j in range(NCHUNK):
        b = j % NBUF
        if j % 2 == 0:
            gathers[j].wait()
        else:
            buf_free(j)
            if j + 1 < NCHUNK:
                buf_free(j + 1)
                gathers[j + 1] = dma_gather(j + 1)
            compute_gather(j, b)
        stores[j] = pltpu.async_copy(
            rows_v.at[b],
            out_hbm.at[pl.ds(base + j * CHUNK, CHUNK)],
            ssems[b],
        )
    for j in range(NCHUNK - NBUF, NCHUNK):
        if j >= 0 and j not in waited:
            stores[j].wait()


def kernel(atomic_numbers, embedding_weight):
    return _emb_lookup(atomic_numbers.astype(jnp.int32), embedding_weight)
